# baseline (speedup 1.0000x reference)
import functools

import jax
import jax.numpy as jnp
from jax import lax
from jax.experimental import pallas as pl
from jax.experimental.pallas import tpu as pltpu
from jax.experimental.pallas import tpu_sc as plsc

NC = 2
NS = 16
L = 16
NW = NC * NS
C = 48


def _sc_scores(x_email, x_noun, eli, per_real, per, nchunk):
    total = per_real * NW
    d_model = x_email.shape[1]
    mesh = plsc.VectorSubcoreMesh(core_axis_name="c", subcore_axis_name="s")

    @functools.partial(
        pl.kernel,
        mesh=mesh,
        compiler_params=pltpu.CompilerParams(use_tc_tiling_on_sc=False,
                                             needs_layout_passes=False),
        out_type=jax.ShapeDtypeStruct((total,), jnp.float32),
        scratch_types=[
            pltpu.VMEM((per,), jnp.int32),
            pltpu.VMEM((per,), jnp.int32),
            pltpu.VMEM((per,), jnp.float32),
            pltpu.VMEM((C, d_model), jnp.bfloat16),
            pltpu.VMEM((C, d_model), jnp.bfloat16),
            pltpu.VMEM((C, d_model), jnp.bfloat16),
            pltpu.VMEM((C, d_model), jnp.bfloat16),
            pltpu.VMEM((L, L + 1), jnp.float32),
            pltpu.VMEM_SHARED(x_email.shape, jnp.bfloat16),
            pltpu.SemaphoreType.DMA,
            pltpu.SemaphoreType.DMA,
            pltpu.SemaphoreType.DMA,
            pltpu.SemaphoreType.DMA,
        ],
    )
    def k(xe_hbm, xn_hbm, eli_hbm, out_hbm,
          i0_v, i1_v, out_v, a0, b0, a1, b1, s_ref, xe_sp,
          sa0, sb0, sa1, sb1):
        wid = lax.axis_index("s") * NC + lax.axis_index("c")
        base = wid * per_real
        sid = lax.axis_index("s")
        rows_per_tile = xe_hbm.shape[0] // NS
        pltpu.sync_copy(
            xe_hbm.at[pl.ds(sid * rows_per_tile, rows_per_tile)],
            xe_sp.at[pl.ds(sid * rows_per_tile, rows_per_tile)])
        pltpu.sync_copy(eli_hbm.at[0, pl.ds(base, per_real)],
                        i0_v.at[pl.ds(0, per_real)])
        pltpu.sync_copy(eli_hbm.at[1, pl.ds(base, per_real)],
                        i1_v.at[pl.ds(0, per_real)])
        lane = lax.iota(jnp.int32, L)
        zero = jnp.zeros((L,), jnp.float32)
        zero_i = jnp.zeros((L,), jnp.int32)
        for pad_off in range(per_real, per, L):
            o = min(pad_off, per - L)
            i0_v[pl.ds(o, L)] = zero_i
            i1_v[pl.ds(o, L)] = zero_i

        def copies(it, buf_a, buf_b, sem_a, sem_b):
            off = it * C
            return (
                pltpu.make_async_copy(
                    xe_sp.at[i0_v.at[pl.ds(off, C)]], buf_a, sem_a),
                pltpu.make_async_copy(
                    xn_hbm.at[i1_v.at[pl.ds(off, C)]], buf_b, sem_b),
            )

        def start(it, buf_a, buf_b, sem_a, sem_b):
            ca, cb = copies(it, buf_a, buf_b, sem_a, sem_b)
            ca.start()
            cb.start()

        def compute(it, buf_a, buf_b, sem_a, sem_b):
            ca, cb = copies(it, buf_a, buf_b, sem_a, sem_b)
            ca.wait()
            cb.wait()
            off = it * C

            def group(g, carry):
                e0 = g * L
                for l in range(L):
                    e = e0 + l
                    acc = zero
                    for j in range(d_model // (2 * L)):
                        ra = buf_a[e, pl.ds(j * 2 * L, 2 * L)]
                        rb = buf_b[e, pl.ds(j * 2 * L, 2 * L)]
                        p_ev, p_od = plsc.unpack(
                            ra * rb, format=plsc.PackFormat.INTERLEAVED,
                            preferred_element_type=jnp.float32)
                        acc = acc + (p_ev + p_od)
                    s_ref[l, pl.ds(0, L)] = acc
                tot = zero
                for k in range(L):
                    cols = jnp.full((L,), k, jnp.int32)
                    tot = tot + plsc.load_gather(s_ref, [lane, cols])
                out_v[pl.ds(off + e0, L)] = tot
                return carry

            lax.fori_loop(0, C // L, group, 0)

        plsc.subcore_barrier()
        start(0, a0, b0, sa0, sb0)
        npair = nchunk // 2

        def pair(p, carry):
            it0 = 2 * p
            start(it0 + 1, a1, b1, sa1, sb1)
            compute(it0, a0, b0, sa0, sb0)

            @pl.when(it0 + 2 < nchunk)
            def _():
                start(it0 + 2, a0, b0, sa0, sb0)

            compute(it0 + 1, a1, b1, sa1, sb1)
            return carry

        lax.fori_loop(0, npair, pair, 0)
        if nchunk % 2:
            compute(nchunk - 1, a0, b0, sa0, sb0)
        pltpu.sync_copy(out_v.at[pl.ds(0, per_real)],
                        out_hbm.at[pl.ds(base, per_real)])

    return k(x_email, x_noun, eli)


def kernel(x_email, x_noun, edge_label_index):
    n_edges = edge_label_index.shape[1]
    x_email = x_email.astype(jnp.bfloat16)
    x_noun = x_noun.astype(jnp.bfloat16)
    per_real = n_edges // NW
    per = -(-per_real // C) * C
    eli = edge_label_index.astype(jnp.int32)
    return _sc_scores(x_email, x_noun, eli, per_real, per, per // C)


# speedup vs baseline: 1.0028x; 1.0028x over previous
import functools

import jax
import jax.numpy as jnp
from jax import lax
from jax.experimental import pallas as pl
from jax.experimental.pallas import tpu as pltpu
from jax.experimental.pallas import tpu_sc as plsc

NC = 2
NS = 16
L = 16
NW = NC * NS
C = 48


def _sc_scores(x_email, x_noun, eli, per_real, per, nchunk):
    total = per_real * NW
    d_model = x_email.shape[1]
    mesh = plsc.VectorSubcoreMesh(core_axis_name="c", subcore_axis_name="s")

    @functools.partial(
        pl.kernel,
        mesh=mesh,
        compiler_params=pltpu.CompilerParams(use_tc_tiling_on_sc=False,
                                             needs_layout_passes=False),
        out_type=jax.ShapeDtypeStruct((total,), jnp.float32),
        scratch_types=[
            pltpu.VMEM((per,), jnp.int32),
            pltpu.VMEM((per,), jnp.int32),
            pltpu.VMEM((per,), jnp.float32),
            pltpu.VMEM((C, d_model), jnp.bfloat16),
            pltpu.VMEM((C, d_model), jnp.bfloat16),
            pltpu.VMEM((C, d_model), jnp.bfloat16),
            pltpu.VMEM((C, d_model), jnp.bfloat16),
            pltpu.VMEM((L, L + 1), jnp.float32),
            pltpu.VMEM_SHARED(x_email.shape, jnp.bfloat16),
            pltpu.SemaphoreType.DMA,
            pltpu.SemaphoreType.DMA,
            pltpu.SemaphoreType.DMA,
            pltpu.SemaphoreType.DMA,
        ],
    )
    def k(xe_hbm, xn_hbm, eli_hbm, out_hbm,
          i0_v, i1_v, out_v, a0, b0, a1, b1, s_ref, xe_sp,
          sa0, sb0, sa1, sb1):
        wid = lax.axis_index("s") * NC + lax.axis_index("c")
        base = wid * per_real
        sid = lax.axis_index("s")
        pltpu.sync_copy(eli_hbm.at[0, pl.ds(base, per_real)],
                        i0_v.at[pl.ds(0, per_real)])
        pltpu.sync_copy(eli_hbm.at[1, pl.ds(base, per_real)],
                        i1_v.at[pl.ds(0, per_real)])
        lane = lax.iota(jnp.int32, L)
        zero = jnp.zeros((L,), jnp.float32)
        zero_i = jnp.zeros((L,), jnp.int32)
        for pad_off in range(per_real, per, L):
            o = min(pad_off, per - L)
            i0_v[pl.ds(o, L)] = zero_i
            i1_v[pl.ds(o, L)] = zero_i

        def copies(it, buf_a, buf_b, sem_a, sem_b):
            off = it * C
            return (
                pltpu.make_async_copy(
                    xe_sp.at[i0_v.at[pl.ds(off, C)]], buf_a, sem_a),
                pltpu.make_async_copy(
                    xn_hbm.at[i1_v.at[pl.ds(off, C)]], buf_b, sem_b),
            )

        def start(it, buf_a, buf_b, sem_a, sem_b):
            ca, cb = copies(it, buf_a, buf_b, sem_a, sem_b)
            ca.start()
            cb.start()

        def compute(it, buf_a, buf_b, sem_a, sem_b):
            ca, cb = copies(it, buf_a, buf_b, sem_a, sem_b)
            ca.wait()
            cb.wait()
            off = it * C

            def group(g, carry):
                e0 = g * L
                for l in range(L):
                    e = e0 + l
                    acc = zero
                    for j in range(d_model // (2 * L)):
                        ra = buf_a[e, pl.ds(j * 2 * L, 2 * L)]
                        rb = buf_b[e, pl.ds(j * 2 * L, 2 * L)]
                        p_ev, p_od = plsc.unpack(
                            ra * rb, format=plsc.PackFormat.INTERLEAVED,
                            preferred_element_type=jnp.float32)
                        acc = acc + (p_ev + p_od)
                    s_ref[l, pl.ds(0, L)] = acc
                tot = zero
                for k in range(L):
                    cols = jnp.full((L,), k, jnp.int32)
                    tot = tot + plsc.load_gather(s_ref, [lane, cols])
                out_v[pl.ds(off + e0, L)] = tot
                return carry

            lax.fori_loop(0, C // L, group, 0)

        ca0, cb0 = copies(0, a0, b0, sa0, sb0)
        cb0.start()
        rows_per_tile = xe_hbm.shape[0] // NS
        pltpu.sync_copy(
            xe_hbm.at[pl.ds(sid * rows_per_tile, rows_per_tile)],
            xe_sp.at[pl.ds(sid * rows_per_tile, rows_per_tile)])
        plsc.subcore_barrier()
        ca0.start()
        npair = nchunk // 2

        def pair(p, carry):
            it0 = 2 * p
            start(it0 + 1, a1, b1, sa1, sb1)
            compute(it0, a0, b0, sa0, sb0)

            @pl.when(it0 + 2 < nchunk)
            def _():
                start(it0 + 2, a0, b0, sa0, sb0)

            compute(it0 + 1, a1, b1, sa1, sb1)
            return carry

        lax.fori_loop(0, npair, pair, 0)
        if nchunk % 2:
            compute(nchunk - 1, a0, b0, sa0, sb0)
        pltpu.sync_copy(out_v.at[pl.ds(0, per_real)],
                        out_hbm.at[pl.ds(base, per_real)])

    return k(x_email, x_noun, eli)


def kernel(x_email, x_noun, edge_label_index):
    n_edges = edge_label_index.shape[1]
    x_email = x_email.astype(jnp.bfloat16)
    x_noun = x_noun.astype(jnp.bfloat16)
    per_real = n_edges // NW
    per = -(-per_real // C) * C
    eli = edge_label_index.astype(jnp.int32)
    return _sc_scores(x_email, x_noun, eli, per_real, per, per // C)
